# baseline (speedup 1.0000x reference)
import math

import jax
import jax.numpy as jnp
from jax import lax
from jax.experimental import pallas as pl
from jax.experimental.pallas import tpu as pltpu
from jax.experimental.pallas import tpu_sc as plsc

NUM_EXPERTS = 16
TOPK = 256
LANES = 16
NCORES = 2


def _gate_body(x_ref, gw_ref, pt_ref):
    xb = x_ref[0]
    gw = gw_ref[...]
    s = lax.dot_general(gw, xb, (((1,), (1,)), ((), ())),
                        preferred_element_type=jnp.float32)
    m = jnp.max(s, axis=0, keepdims=True)
    e = jnp.exp(s - m)
    p = e / jnp.sum(e, axis=0, keepdims=True)
    pt_ref[0] = lax.bitcast_convert_type(p, jnp.int32)


def _srl31(x):
    return lax.shift_right_logical(x, 31)


def _routing_body(pt_hbm, x_hbm, idx_hbm, pv_hbm, xin_hbm,
                  prow, idxv, pvv, gidx, gbuf, idx_smem, pv_smem, sem):
    c = lax.axis_index("c")
    s = lax.axis_index("s")
    r = s * NCORES + c
    T = prow.shape[0]
    NV = T // LANES
    b = r // NUM_EXPERTS

    pltpu.sync_copy(pt_hbm.at[r], prow)

    def count_gt(mid):
        tfv = jnp.full((LANES,), mid, jnp.int32)

        def cbody(i, acc):
            a = acc
            for u in range(4):
                v = prow[pl.ds((i * 4 + u) * LANES, LANES)]
                a = a + _srl31(tfv - v)
            return a

        acc = lax.fori_loop(0, NV // 4, cbody, jnp.zeros((LANES,), jnp.int32))
        tot = acc[0]
        for j in range(1, LANES):
            tot = tot + acc[j]
        return tot

    def bs_body(_, carry):
        lo, hi = carry
        mid = lo + ((hi - lo) >> 1)
        big = count_gt(mid) >= TOPK
        return jnp.where(big, mid, lo), jnp.where(big, hi, mid)

    lo, hi = lax.fori_loop(0, 31, bs_body,
                           (jnp.int32(0), jnp.int32(0x3F800001)))
    tf_s = hi
    n_gt = count_gt(tf_s)
    rem = TOPK - n_gt

    TRASH = jnp.int32(TOPK)

    def comp_body(i, carry):
        ptr, eq_taken = carry
        v = prow[pl.ds(i * LANES, LANES)]
        for j in range(LANES):
            val = v[j]
            take_eq = (val == tf_s) & (eq_taken < rem)
            take = (val > tf_s) | take_eq
            pos = jnp.where(take, ptr, TRASH)
            idx_smem[pos] = i * LANES + j
            pv_smem[pos] = val
            ptr = ptr + take.astype(jnp.int32)
            eq_taken = eq_taken + take_eq.astype(jnp.int32)
        return ptr, eq_taken

    lax.fori_loop(0, NV, comp_body, (jnp.int32(0), jnp.int32(0)))

    iota = lax.iota(jnp.int32, LANES)
    ones = [1 - _srl31((iota - j) | (j - iota)) for j in range(LANES)]
    boff = b * T
    for k2 in range(TOPK // LANES):
        w = jnp.zeros((LANES,), jnp.int32)
        p = jnp.zeros((LANES,), jnp.int32)
        for j in range(LANES):
            w = w + jnp.full((LANES,), idx_smem[k2 * LANES + j],
                             jnp.int32) * ones[j]
            p = p + jnp.full((LANES,), pv_smem[k2 * LANES + j],
                             jnp.int32) * ones[j]
        idxv[pl.ds(k2 * LANES, LANES)] = w
        pvv[pl.ds(k2 * LANES, LANES)] = p
        gidx[k2 // 4, pl.ds((k2 % 4) * LANES, LANES)] = w + boff

    pltpu.sync_copy(idxv, idx_hbm.at[r])
    pltpu.sync_copy(pvv, pv_hbm.at[r])
    for j in range(4):
        pltpu.async_copy(x_hbm.at[gidx.at[j]], gbuf, sem).wait()
        pltpu.sync_copy(gbuf, xin_hbm.at[pl.ds(r * TOPK + j * 64, 64)])


def _ffn_body(xin_ref, pv_ref, w1_ref, w2_ref, y_ref):
    xin = xin_ref[0]
    pv = lax.bitcast_convert_type(pv_ref[0, 0], jnp.float32)

    h = lax.dot_general(xin.astype(jnp.bfloat16),
                        w1_ref[0].astype(jnp.bfloat16),
                        (((1,), (0,)), ((), ())),
                        preferred_element_type=jnp.float32)
    h = 0.5 * h * (1.0 + lax.erf(h * (1.0 / math.sqrt(2.0))))
    y = lax.dot_general(h.astype(jnp.bfloat16),
                        w2_ref[0].astype(jnp.bfloat16),
                        (((1,), (0,)), ((), ())),
                        preferred_element_type=jnp.float32)
    y_ref[0] = (y * pv[:, None]).astype(jnp.bfloat16)


def _scatter_body(y_ref, idx_ref, out_ref):
    T = out_ref.shape[1]
    idxv = idx_ref[0, 0]
    EK = idxv.shape[0]
    onehot = (lax.broadcasted_iota(jnp.int32, (EK, T), 1) == idxv[:, None])
    oh_bf = onehot.astype(jnp.bfloat16)
    out_ref[0] = lax.dot_general(oh_bf, y_ref[0], (((0,), (0,)), ((), ())),
                                 preferred_element_type=jnp.float32)


def kernel(x, gate_w, w1, w2):
    B, T, C = x.shape
    E = gate_w.shape[0]
    H = w1.shape[2]
    K = TOPK
    R = B * E

    pt = pl.pallas_call(
        _gate_body,
        grid=(B,),
        in_specs=[
            pl.BlockSpec((1, T, C), lambda b: (b, 0, 0)),
            pl.BlockSpec((E, C), lambda b: (0, 0)),
        ],
        out_specs=pl.BlockSpec((1, E, T), lambda b: (b, 0, 0)),
        out_shape=jax.ShapeDtypeStruct((B, E, T), jnp.int32),
    )(x, gate_w)

    mesh = plsc.VectorSubcoreMesh(core_axis_name="c", subcore_axis_name="s")
    routing = pl.kernel(
        _routing_body,
        mesh=mesh,
        out_type=[
            jax.ShapeDtypeStruct((R, K), jnp.int32),
            jax.ShapeDtypeStruct((R, K), jnp.int32),
            jax.ShapeDtypeStruct((R * K, C), jnp.float32),
        ],
        scratch_types=[
            pltpu.VMEM((T,), jnp.int32),
            pltpu.VMEM((K,), jnp.int32),
            pltpu.VMEM((K,), jnp.int32),
            pltpu.VMEM((4, 64), jnp.int32),
            pltpu.VMEM((64, C), jnp.float32),
            pltpu.SMEM((K + LANES,), jnp.int32),
            pltpu.SMEM((K + LANES,), jnp.int32),
            pltpu.SemaphoreType.DMA,
        ],
    )
    idx, pvb, xin = routing(pt.reshape(R, T), x.reshape(B * T, C))

    y = pl.pallas_call(
        _ffn_body,
        grid=(E, B),
        in_specs=[
            pl.BlockSpec((1, K, C), lambda e, b: (b * NUM_EXPERTS + e, 0, 0)),
            pl.BlockSpec((1, 1, K), lambda e, b: (b * NUM_EXPERTS + e, 0, 0)),
            pl.BlockSpec((1, C, H), lambda e, b: (e, 0, 0)),
            pl.BlockSpec((1, H, C), lambda e, b: (e, 0, 0)),
        ],
        out_specs=pl.BlockSpec((1, K, C), lambda e, b: (b * NUM_EXPERTS + e, 0, 0)),
        out_shape=jax.ShapeDtypeStruct((R, K, C), jnp.bfloat16),
    )(xin.reshape(R, K, C), pvb.reshape(R, 1, K), w1, w2)

    out = pl.pallas_call(
        _scatter_body,
        grid=(B,),
        in_specs=[
            pl.BlockSpec((1, E * K, C), lambda b: (b, 0, 0)),
            pl.BlockSpec((1, 1, E * K), lambda b: (b, 0, 0)),
        ],
        out_specs=pl.BlockSpec((1, T, C), lambda b: (b, 0, 0)),
        out_shape=jax.ShapeDtypeStruct((B, T, C), jnp.float32),
    )(y.reshape(B, E * K, C), idx.reshape(B, 1, E * K))
    return out


# speedup vs baseline: 1.0874x; 1.0874x over previous
import math

import jax
import jax.numpy as jnp
from jax import lax
from jax.experimental import pallas as pl
from jax.experimental.pallas import tpu as pltpu
from jax.experimental.pallas import tpu_sc as plsc

NUM_EXPERTS = 16
TOPK = 256
LANES = 16
NCORES = 2


def _gate_body(x_ref, gw_ref, pt_ref, xbf_ref):
    xb = x_ref[0]
    gw = gw_ref[...]
    s = lax.dot_general(gw, xb, (((1,), (1,)), ((), ())),
                        preferred_element_type=jnp.float32)
    m = jnp.max(s, axis=0, keepdims=True)
    e = jnp.exp(s - m)
    p = e / jnp.sum(e, axis=0, keepdims=True)
    pt_ref[0] = lax.bitcast_convert_type(p, jnp.int32)
    xbf_ref[0] = xb.astype(jnp.bfloat16)


def _srl31(x):
    return lax.shift_right_logical(x, 31)


def _routing_body(pt_hbm, idx_hbm, pv_hbm, prow, idxv, pvv, idx_smem, pv_smem):
    c = lax.axis_index("c")
    s = lax.axis_index("s")
    r = s * NCORES + c
    T = prow.shape[0]
    NV = T // LANES
    b = r // NUM_EXPERTS
    e = r % NUM_EXPERTS

    pltpu.sync_copy(pt_hbm.at[r], prow)

    def count_gt(mid):
        tfv = jnp.full((LANES,), mid, jnp.int32)

        def cbody(i, acc):
            a = acc
            for u in range(4):
                v = prow[pl.ds((i * 4 + u) * LANES, LANES)]
                a = a + _srl31(tfv - v)
            return a

        acc = lax.fori_loop(0, NV // 4, cbody, jnp.zeros((LANES,), jnp.int32))
        tot = acc[0]
        for j in range(1, LANES):
            tot = tot + acc[j]
        return tot

    def bs_body(_, carry):
        lo, hi = carry
        mid = lo + ((hi - lo) >> 1)
        big = count_gt(mid) >= TOPK
        return jnp.where(big, mid, lo), jnp.where(big, hi, mid)

    lo, hi = lax.fori_loop(0, 31, bs_body,
                           (jnp.int32(0), jnp.int32(0x3F800001)))
    tf_s = hi
    n_gt = count_gt(tf_s)
    rem = TOPK - n_gt

    TRASH = jnp.int32(TOPK)

    def comp_body(i, carry):
        ptr, eq_taken = carry
        v = prow[pl.ds(i * LANES, LANES)]
        for j in range(LANES):
            val = v[j]
            take_eq = (val == tf_s) & (eq_taken < rem)
            take = (val > tf_s) | take_eq
            pos = jnp.where(take, ptr, TRASH)
            idx_smem[pos] = i * LANES + j
            pv_smem[pos] = val
            ptr = ptr + take.astype(jnp.int32)
            eq_taken = eq_taken + take_eq.astype(jnp.int32)
        return ptr, eq_taken

    lax.fori_loop(0, NV, comp_body, (jnp.int32(0), jnp.int32(0)))

    iota = lax.iota(jnp.int32, LANES)
    ones = [1 - _srl31((iota - j) | (j - iota)) for j in range(LANES)]
    for k2 in range(TOPK // LANES):
        w = jnp.zeros((LANES,), jnp.int32)
        p = jnp.zeros((LANES,), jnp.int32)
        for j in range(LANES):
            w = w + jnp.full((LANES,), idx_smem[k2 * LANES + j],
                             jnp.int32) * ones[j]
            p = p + jnp.full((LANES,), pv_smem[k2 * LANES + j],
                             jnp.int32) * ones[j]
        idxv[pl.ds(k2 * LANES, LANES)] = w
        pvv[pl.ds(k2 * LANES, LANES)] = p

    row2 = e * (idx_hbm.shape[0] // NUM_EXPERTS) + b
    pltpu.sync_copy(idxv, idx_hbm.at[row2])
    pltpu.sync_copy(pvv, pv_hbm.at[row2])


def _ffn_body(xbf_ref, idx_ref, pv_ref, w1_ref, w2_ref, out_ref):
    e = pl.program_id(0)

    @pl.when(e == 0)
    def _():
        out_ref[...] = jnp.zeros_like(out_ref)

    B, T, C = out_ref.shape
    K = TOPK
    w1b = w1_ref[0].astype(jnp.bfloat16)
    w2b = w2_ref[0].astype(jnp.bfloat16)
    iota_t = lax.broadcasted_iota(jnp.int32, (1, T), 1)
    for b in range(B):
        idxv = idx_ref[0, 0, pl.ds(b * K, K)]
        pv = lax.bitcast_convert_type(pv_ref[0, 0, pl.ds(b * K, K)],
                                      jnp.float32)
        oh_bf = (iota_t == idxv[:, None]).astype(jnp.bfloat16)
        xg = lax.dot_general(oh_bf, xbf_ref[b], (((1,), (0,)), ((), ())),
                             preferred_element_type=jnp.float32)
        h = lax.dot_general(xg.astype(jnp.bfloat16), w1b,
                            (((1,), (0,)), ((), ())),
                            preferred_element_type=jnp.float32)
        h = 0.5 * h * (1.0 + lax.erf(h * (1.0 / math.sqrt(2.0))))
        y = lax.dot_general(h.astype(jnp.bfloat16), w2b,
                            (((1,), (0,)), ((), ())),
                            preferred_element_type=jnp.float32)
        y = y * pv[:, None]
        out_ref[b] += lax.dot_general(oh_bf, y.astype(jnp.bfloat16),
                                      (((0,), (0,)), ((), ())),
                                      preferred_element_type=jnp.float32)


def kernel(x, gate_w, w1, w2):
    B, T, C = x.shape
    E = gate_w.shape[0]
    H = w1.shape[2]
    K = TOPK
    R = B * E

    pt, xbf = pl.pallas_call(
        _gate_body,
        grid=(B,),
        in_specs=[
            pl.BlockSpec((1, T, C), lambda b: (b, 0, 0)),
            pl.BlockSpec((E, C), lambda b: (0, 0)),
        ],
        out_specs=[
            pl.BlockSpec((1, E, T), lambda b: (b, 0, 0)),
            pl.BlockSpec((1, T, C), lambda b: (b, 0, 0)),
        ],
        out_shape=[
            jax.ShapeDtypeStruct((B, E, T), jnp.int32),
            jax.ShapeDtypeStruct((B, T, C), jnp.bfloat16),
        ],
    )(x, gate_w)

    mesh = plsc.VectorSubcoreMesh(core_axis_name="c", subcore_axis_name="s")
    routing = pl.kernel(
        _routing_body,
        mesh=mesh,
        out_type=[
            jax.ShapeDtypeStruct((R, K), jnp.int32),
            jax.ShapeDtypeStruct((R, K), jnp.int32),
        ],
        scratch_types=[
            pltpu.VMEM((T,), jnp.int32),
            pltpu.VMEM((K,), jnp.int32),
            pltpu.VMEM((K,), jnp.int32),
            pltpu.SMEM((K + LANES,), jnp.int32),
            pltpu.SMEM((K + LANES,), jnp.int32),
        ],
    )
    idx, pvb = routing(pt.reshape(R, T))

    out = pl.pallas_call(
        _ffn_body,
        grid=(E,),
        in_specs=[
            pl.BlockSpec((B, T, C), lambda e: (0, 0, 0)),
            pl.BlockSpec((1, 1, B * K), lambda e: (e, 0, 0)),
            pl.BlockSpec((1, 1, B * K), lambda e: (e, 0, 0)),
            pl.BlockSpec((1, C, H), lambda e: (e, 0, 0)),
            pl.BlockSpec((1, H, C), lambda e: (e, 0, 0)),
        ],
        out_specs=pl.BlockSpec((B, T, C), lambda e: (0, 0, 0)),
        out_shape=jax.ShapeDtypeStruct((B, T, C), jnp.float32),
    )(xbf, idx.reshape(E, 1, B * K), pvb.reshape(E, 1, B * K), w1, w2)
    return out


# speedup vs baseline: 1.2161x; 1.1183x over previous
import math

import jax
import jax.numpy as jnp
from jax import lax
from jax.experimental import pallas as pl
from jax.experimental.pallas import tpu as pltpu
from jax.experimental.pallas import tpu_sc as plsc

NUM_EXPERTS = 16
TOPK = 256
LANES = 16
NCORES = 2


def _gate_body(x_ref, gw_ref, pt_ref, xbf_ref):
    xb = x_ref[0]
    gw = gw_ref[...]
    s = lax.dot_general(gw, xb, (((1,), (1,)), ((), ())),
                        preferred_element_type=jnp.float32)
    m = jnp.max(s, axis=0, keepdims=True)
    e = jnp.exp(s - m)
    p = e / jnp.sum(e, axis=0, keepdims=True)
    pt_ref[0] = lax.bitcast_convert_type(p, jnp.int32)
    xbf_ref[0] = xb.astype(jnp.bfloat16)


def _srl31(x):
    return lax.shift_right_logical(x, 31)


def _routing_body(pt_hbm, idx_hbm, pv_hbm, prow, idxv, pvv, idx_smem, pv_smem):
    c = lax.axis_index("c")
    s = lax.axis_index("s")
    r = s * NCORES + c
    T = prow.shape[0]
    NV = T // LANES
    b = r // NUM_EXPERTS
    e = r % NUM_EXPERTS

    pltpu.sync_copy(pt_hbm.at[r], prow)

    def count_gt(mid):
        tfv = jnp.full((LANES,), mid, jnp.int32)

        def cbody(i, acc):
            a = acc
            for u in range(4):
                v = prow[pl.ds((i * 4 + u) * LANES, LANES)]
                a = a + _srl31(tfv - v)
            return a

        acc = lax.fori_loop(0, NV // 4, cbody, jnp.zeros((LANES,), jnp.int32))
        tot = acc[0]
        for j in range(1, LANES):
            tot = tot + acc[j]
        return tot

    def bs_body(_, carry):
        lo, hi = carry
        mid = lo + ((hi - lo) >> 1)
        big = count_gt(mid) >= TOPK
        return jnp.where(big, mid, lo), jnp.where(big, hi, mid)

    lo, hi = lax.fori_loop(0, 31, bs_body,
                           (jnp.int32(0), jnp.int32(0x3F800001)))
    tf_s = hi
    n_gt = count_gt(tf_s)
    rem = TOPK - n_gt

    TRASH = jnp.int32(TOPK)

    def comp_body(i, carry):
        ptr, eq_taken = carry
        v = prow[pl.ds(i * LANES, LANES)]
        for j in range(LANES):
            val = v[j]
            take_eq = (val == tf_s) & (eq_taken < rem)
            take = (val > tf_s) | take_eq
            pos = jnp.where(take, ptr, TRASH)
            idx_smem[pos] = i * LANES + j
            pv_smem[pos] = val
            ptr = ptr + take.astype(jnp.int32)
            eq_taken = eq_taken + take_eq.astype(jnp.int32)
        return ptr, eq_taken

    lax.fori_loop(0, NV, comp_body, (jnp.int32(0), jnp.int32(0)))

    iota = lax.iota(jnp.int32, LANES)
    ones = [1 - _srl31((iota - j) | (j - iota)) for j in range(LANES)]
    for k2 in range(TOPK // LANES):
        w = jnp.zeros((LANES,), jnp.int32)
        p = jnp.zeros((LANES,), jnp.int32)
        for j in range(LANES):
            w = w + jnp.full((LANES,), idx_smem[k2 * LANES + j],
                             jnp.int32) * ones[j]
            p = p + jnp.full((LANES,), pv_smem[k2 * LANES + j],
                             jnp.int32) * ones[j]
        idxv[pl.ds(k2 * LANES, LANES)] = w
        pvv[pl.ds(k2 * LANES, LANES)] = p

    row2 = e * (idx_hbm.shape[0] // NUM_EXPERTS) + b
    pltpu.sync_copy(idxv, idx_hbm.at[row2])
    pltpu.sync_copy(pvv, pv_hbm.at[row2])


def _ffn_body(xbf_ref, idx_ref, pv_ref, w1_ref, w2_ref, out_ref):
    e = pl.program_id(0)

    @pl.when(e == 0)
    def _():
        out_ref[...] = jnp.zeros_like(out_ref)

    B, T, C = out_ref.shape
    K = TOPK
    w1b = w1_ref[0].astype(jnp.bfloat16)
    w2b = w2_ref[0].astype(jnp.bfloat16)
    iota_t = lax.broadcasted_iota(jnp.int32, (1, T), 1)
    ohs = []
    xgs = []
    for b in range(B):
        idxv = idx_ref[0, 0, pl.ds(b * K, K)]
        oh_bf = (iota_t == idxv[:, None]).astype(jnp.bfloat16)
        ohs.append(oh_bf)
        xgs.append(
            lax.dot_general(oh_bf, xbf_ref[b], (((1,), (0,)), ((), ())),
                            preferred_element_type=jnp.float32))
    xg = jnp.concatenate(xgs, axis=0)
    h = lax.dot_general(xg.astype(jnp.bfloat16), w1b,
                        (((1,), (0,)), ((), ())),
                        preferred_element_type=jnp.float32)
    h = 0.5 * h * (1.0 + lax.erf(h * (1.0 / math.sqrt(2.0))))
    y = lax.dot_general(h.astype(jnp.bfloat16), w2b,
                        (((1,), (0,)), ((), ())),
                        preferred_element_type=jnp.float32)
    pv = lax.bitcast_convert_type(pv_ref[0, 0], jnp.float32)
    y = (y * pv[:, None]).astype(jnp.bfloat16)
    for b in range(B):
        out_ref[b] += lax.dot_general(ohs[b], y[b * K:(b + 1) * K],
                                      (((0,), (0,)), ((), ())),
                                      preferred_element_type=jnp.float32)


def kernel(x, gate_w, w1, w2):
    B, T, C = x.shape
    E = gate_w.shape[0]
    H = w1.shape[2]
    K = TOPK
    R = B * E

    pt, xbf = pl.pallas_call(
        _gate_body,
        grid=(B,),
        in_specs=[
            pl.BlockSpec((1, T, C), lambda b: (b, 0, 0)),
            pl.BlockSpec((E, C), lambda b: (0, 0)),
        ],
        out_specs=[
            pl.BlockSpec((1, E, T), lambda b: (b, 0, 0)),
            pl.BlockSpec((1, T, C), lambda b: (b, 0, 0)),
        ],
        out_shape=[
            jax.ShapeDtypeStruct((B, E, T), jnp.int32),
            jax.ShapeDtypeStruct((B, T, C), jnp.bfloat16),
        ],
    )(x, gate_w)

    mesh = plsc.VectorSubcoreMesh(core_axis_name="c", subcore_axis_name="s")
    routing = pl.kernel(
        _routing_body,
        mesh=mesh,
        out_type=[
            jax.ShapeDtypeStruct((R, K), jnp.int32),
            jax.ShapeDtypeStruct((R, K), jnp.int32),
        ],
        scratch_types=[
            pltpu.VMEM((T,), jnp.int32),
            pltpu.VMEM((K,), jnp.int32),
            pltpu.VMEM((K,), jnp.int32),
            pltpu.SMEM((K + LANES,), jnp.int32),
            pltpu.SMEM((K + LANES,), jnp.int32),
        ],
    )
    idx, pvb = routing(pt.reshape(R, T))

    out = pl.pallas_call(
        _ffn_body,
        grid=(E,),
        in_specs=[
            pl.BlockSpec((B, T, C), lambda e: (0, 0, 0)),
            pl.BlockSpec((1, 1, B * K), lambda e: (e, 0, 0)),
            pl.BlockSpec((1, 1, B * K), lambda e: (e, 0, 0)),
            pl.BlockSpec((1, C, H), lambda e: (e, 0, 0)),
            pl.BlockSpec((1, H, C), lambda e: (e, 0, 0)),
        ],
        out_specs=pl.BlockSpec((B, T, C), lambda e: (0, 0, 0)),
        out_shape=jax.ShapeDtypeStruct((B, T, C), jnp.float32),
    )(xbf, idx.reshape(E, 1, B * K), pvb.reshape(E, 1, B * K), w1, w2)
    return out
